# Initial kernel scaffold; baseline (speedup 1.0000x reference)
import functools

import numpy as np
import jax
import jax.numpy as jnp
from jax import lax
from jax.experimental import pallas as pl
from jax.experimental.pallas import tpu as pltpu
from jax.experimental.pallas import tpu_sc as plsc

N = 10000
E = 160000
IN = 8
SH = 4
OUT = 16
F = 64
H = 64
K = IN * SH
WID = 2 * OUT
SCALE = 1.0 / float(np.sqrt(IN * SH))

NC, NS = 2, 16
NW = NC * NS
EPW = E // NW
NPT = N // NS
CH = 1000

BE = 640

_c = np.arange(K * OUT)
_RX = ((_c // (SH * OUT))[None, :] == np.arange(2 * IN)[:, None]).astype(np.float32)
_RS = (((_c // OUT) % SH)[None, :] == np.arange(SH)[:, None]).astype(np.float32)


def _sc_gather(table, idx):
    mesh = plsc.VectorSubcoreMesh(core_axis_name="c", subcore_axis_name="s")

    @functools.partial(
        pl.kernel, mesh=mesh,
        out_type=jax.ShapeDtypeStruct((E, 2 * IN), jnp.float32),
        scratch_types=[
            pltpu.VMEM((EPW,), jnp.int32),
            pltpu.VMEM((EPW, 2 * IN), jnp.float32),
            pltpu.SemaphoreType.DMA,
        ],
    )
    def gk(table_hbm, idx_hbm, out_hbm, idx_v, rows_v, sem):
        wid = lax.axis_index("s") * NC + lax.axis_index("c")
        base = wid * EPW
        pltpu.sync_copy(idx_hbm.at[pl.ds(base, EPW)], idx_v)
        pltpu.async_copy(table_hbm.at[idx_v], rows_v, sem).wait()
        pltpu.sync_copy(rows_v, out_hbm.at[pl.ds(base, EPW)])

    return gk(table, idx)


def _sc_scatter(rows, src, zeros):
    mesh = plsc.VectorSubcoreMesh(core_axis_name="c", subcore_axis_name="s")

    @functools.partial(
        pl.kernel, mesh=mesh,
        out_type=jax.ShapeDtypeStruct((NC, N, WID), jnp.float32),
        scratch_types=[
            pltpu.VMEM((CH,), jnp.int32),
            pltpu.VMEM((CH, WID), jnp.float32),
            pltpu.VMEM_SHARED((N, WID), jnp.float32),
            pltpu.SemaphoreType.DMA,
        ],
    )
    def sk(rows_hbm, src_hbm, zeros_hbm, out_hbm, idx_v, rows_v, acc_sh, sem):
        c = lax.axis_index("c")
        s = lax.axis_index("s")
        pltpu.sync_copy(zeros_hbm.at[pl.ds(s * NPT, NPT)],
                        acc_sh.at[pl.ds(s * NPT, NPT)])
        plsc.subcore_barrier()
        base = (s * NC + c) * EPW

        def body(i, carry):
            off = base + i * CH
            pltpu.sync_copy(src_hbm.at[pl.ds(off, CH)], idx_v)
            pltpu.sync_copy(rows_hbm.at[pl.ds(off, CH)], rows_v)
            pltpu.sync_copy(rows_v, acc_sh.at[idx_v], add=True)
            return carry

        lax.fori_loop(0, EPW // CH, body, 0)
        plsc.subcore_barrier()
        pltpu.sync_copy(acc_sh.at[pl.ds(s * NPT, NPT)],
                        out_hbm.at[c, pl.ds(s * NPT, NPT)])

    return sk(rows, src, zeros)


def _tc_main_body(x_ref, sh_ref, ea_ref, w1_ref, b1_ref, w2_ref, b2_ref,
                  rx_ref, rs_ref, out_ref):
    h = jnp.maximum(
        jnp.dot(ea_ref[...], w1_ref[...], preferred_element_type=jnp.float32)
        + b1_ref[...], 0.0)
    w = jnp.dot(h, w2_ref[...], preferred_element_type=jnp.float32) + b2_ref[...]
    xb = jnp.dot(x_ref[...], rx_ref[...], preferred_element_type=jnp.float32)
    sb = jnp.dot(sh_ref[...], rs_ref[...], preferred_element_type=jnp.float32)
    p = xb * sb * w
    t = p[:, :256] + p[:, 256:]
    t = t[:, :128] + t[:, 128:]
    t = t[:, :64] + t[:, 64:]
    t = t[:, :32] + t[:, 32:]
    tp = (t[:, :16] + t[:, 16:]) * SCALE
    out_ref[...] = jnp.concatenate(
        [tp, jnp.ones((tp.shape[0], OUT), jnp.float32)], axis=1)


def _tc_main(x, sh, ea, w1, b1, w2, b2):
    nb = E // BE
    fixed = lambda i: (0, 0)
    return pl.pallas_call(
        _tc_main_body,
        grid=(nb,),
        in_specs=[
            pl.BlockSpec((BE, 2 * IN), lambda i: (i, 0)),
            pl.BlockSpec((BE, SH), lambda i: (i, 0)),
            pl.BlockSpec((BE, F), lambda i: (i, 0)),
            pl.BlockSpec((F, H), fixed),
            pl.BlockSpec((1, H), fixed),
            pl.BlockSpec((H, K * OUT), fixed),
            pl.BlockSpec((1, K * OUT), fixed),
            pl.BlockSpec((2 * IN, K * OUT), fixed),
            pl.BlockSpec((SH, K * OUT), fixed),
        ],
        out_specs=pl.BlockSpec((BE, WID), lambda i: (i, 0)),
        out_shape=jax.ShapeDtypeStruct((E, WID), jnp.float32),
    )(x, sh, ea, w1, b1, w2, b2, jnp.asarray(_RX), jnp.asarray(_RS))


def _tc_norm_body(a0_ref, a1_ref, na_ref, g_ref, b_ref, out_ref):
    a = a0_ref[...] + a1_ref[...]
    sums = a[:, :OUT]
    cnt = a[:, OUT:OUT + 1]
    o = sums / jnp.maximum(cnt, 1.0) + na_ref[...]
    mean = jnp.mean(o, axis=0, keepdims=True)
    var = jnp.mean((o - mean) ** 2, axis=0, keepdims=True)
    out_ref[...] = (o - mean) * lax.rsqrt(var + 1e-5) * g_ref[...] + b_ref[...]


def _tc_norm(a0, a1, na16, gamma, beta):
    return pl.pallas_call(
        _tc_norm_body,
        out_shape=jax.ShapeDtypeStruct((N, OUT), jnp.float32),
    )(a0, a1, na16, gamma, beta)


def kernel(node_attr, edge_index, edge_attr, edge_sh, W1, b1, W2, b2, gamma, beta):
    na16 = jnp.pad(node_attr, ((0, 0), (0, 2 * IN - IN)))
    src = edge_index[0]
    dst = edge_index[1]
    x = _sc_gather(na16, dst)
    rows = _tc_main(x, edge_sh, edge_attr, W1, b1.reshape(1, H), W2,
                    b2.reshape(1, K * OUT))
    acc = _sc_scatter(rows, src, jnp.zeros((N, WID), jnp.float32))
    return _tc_norm(acc[0], acc[1], na16, gamma.reshape(1, OUT),
                    beta.reshape(1, OUT))


# speedup vs baseline: 2.4167x; 2.4167x over previous
import functools

import numpy as np
import jax
import jax.numpy as jnp
from jax import lax
from jax.experimental import pallas as pl
from jax.experimental.pallas import tpu as pltpu
from jax.experimental.pallas import tpu_sc as plsc

N = 10000
E = 160000
IN = 8
SH = 4
OUT = 16
F = 64
H = 64
K = IN * SH
WID = 2 * OUT
SCALE = 1.0 / float(np.sqrt(IN * SH))

NC, NS = 2, 16
NW = NC * NS
EPW = E // NW
NPT = N // NS
CH = 1000

BE = 640

_c = np.arange(K * OUT)
_RX = ((_c // (SH * OUT))[None, :] == np.arange(2 * IN)[:, None]).astype(np.float32)
_RS = (((_c // OUT) % SH)[None, :] == np.arange(SH)[:, None]).astype(np.float32)


def _sc_gather(table, idx):
    mesh = plsc.VectorSubcoreMesh(core_axis_name="c", subcore_axis_name="s")

    @functools.partial(
        pl.kernel, mesh=mesh,
        out_type=jax.ShapeDtypeStruct((E, 2 * IN), jnp.float32),
        compiler_params=pltpu.CompilerParams(use_tc_tiling_on_sc=False),
        scratch_types=[
            pltpu.VMEM((EPW,), jnp.int32),
            pltpu.VMEM((EPW, 2 * IN), jnp.float32),
            pltpu.SemaphoreType.DMA,
        ],
    )
    def gk(table_hbm, idx_hbm, out_hbm, idx_v, rows_v, sem):
        wid = lax.axis_index("s") * NC + lax.axis_index("c")
        base = wid * EPW
        pltpu.sync_copy(idx_hbm.at[pl.ds(base, EPW)], idx_v)
        pltpu.async_copy(table_hbm.at[idx_v], rows_v, sem).wait()
        pltpu.sync_copy(rows_v, out_hbm.at[pl.ds(base, EPW)])

    return gk(table, idx)


def _sc_scatter(rows, src, zeros):
    mesh = plsc.VectorSubcoreMesh(core_axis_name="c", subcore_axis_name="s")

    @functools.partial(
        pl.kernel, mesh=mesh,
        out_type=jax.ShapeDtypeStruct((NC, N, WID), jnp.float32),
        compiler_params=pltpu.CompilerParams(use_tc_tiling_on_sc=False),
        scratch_types=[
            pltpu.VMEM((CH,), jnp.int32),
            pltpu.VMEM((CH, WID), jnp.float32),
            pltpu.VMEM_SHARED((N, WID), jnp.float32),
            pltpu.SemaphoreType.DMA,
        ],
    )
    def sk(rows_hbm, src_hbm, zeros_hbm, out_hbm, idx_v, rows_v, acc_sh, sem):
        c = lax.axis_index("c")
        s = lax.axis_index("s")
        pltpu.sync_copy(zeros_hbm.at[pl.ds(s * NPT, NPT)],
                        acc_sh.at[pl.ds(s * NPT, NPT)])
        plsc.subcore_barrier()
        base = (s * NC + c) * EPW

        def body(i, carry):
            off = base + i * CH
            pltpu.sync_copy(src_hbm.at[pl.ds(off, CH)], idx_v)
            pltpu.sync_copy(rows_hbm.at[pl.ds(off, CH)], rows_v)
            pltpu.sync_copy(rows_v, acc_sh.at[idx_v], add=True)
            return carry

        lax.fori_loop(0, EPW // CH, body, 0)
        plsc.subcore_barrier()
        pltpu.sync_copy(acc_sh.at[pl.ds(s * NPT, NPT)],
                        out_hbm.at[c, pl.ds(s * NPT, NPT)])

    return sk(rows, src, zeros)


def _tc_main_body(x_ref, sh_ref, ea_ref, w1_ref, b1_ref, w2_ref, b2_ref,
                  rx_ref, rs_ref, out_ref):
    h = jnp.maximum(
        jnp.dot(ea_ref[...], w1_ref[...], preferred_element_type=jnp.float32)
        + b1_ref[...], 0.0)
    w = jnp.dot(h, w2_ref[...], preferred_element_type=jnp.float32) + b2_ref[...]
    xb = jnp.dot(x_ref[...], rx_ref[...], preferred_element_type=jnp.float32)
    sb = jnp.dot(sh_ref[...], rs_ref[...], preferred_element_type=jnp.float32)
    p = xb * sb * w
    t = p[:, :256] + p[:, 256:]
    t = t[:, :128] + t[:, 128:]
    t = t[:, :64] + t[:, 64:]
    t = t[:, :32] + t[:, 32:]
    tp = (t[:, :16] + t[:, 16:]) * SCALE
    out_ref[...] = jnp.concatenate(
        [tp, jnp.ones((tp.shape[0], OUT), jnp.float32)], axis=1)


def _tc_main(x, sh, ea, w1, b1, w2, b2):
    nb = E // BE
    fixed = lambda i: (0, 0)
    return pl.pallas_call(
        _tc_main_body,
        grid=(nb,),
        in_specs=[
            pl.BlockSpec((BE, 2 * IN), lambda i: (i, 0)),
            pl.BlockSpec((BE, SH), lambda i: (i, 0)),
            pl.BlockSpec((BE, F), lambda i: (i, 0)),
            pl.BlockSpec((F, H), fixed),
            pl.BlockSpec((1, H), fixed),
            pl.BlockSpec((H, K * OUT), fixed),
            pl.BlockSpec((1, K * OUT), fixed),
            pl.BlockSpec((2 * IN, K * OUT), fixed),
            pl.BlockSpec((SH, K * OUT), fixed),
        ],
        out_specs=pl.BlockSpec((BE, WID), lambda i: (i, 0)),
        out_shape=jax.ShapeDtypeStruct((E, WID), jnp.float32),
    )(x, sh, ea, w1, b1, w2, b2, jnp.asarray(_RX), jnp.asarray(_RS))


def _tc_norm_body(a0_ref, a1_ref, na_ref, g_ref, b_ref, out_ref):
    a = a0_ref[...] + a1_ref[...]
    sums = a[:, :OUT]
    cnt = a[:, OUT:OUT + 1]
    o = sums / jnp.maximum(cnt, 1.0) + na_ref[...]
    mean = jnp.mean(o, axis=0, keepdims=True)
    var = jnp.mean((o - mean) ** 2, axis=0, keepdims=True)
    out_ref[...] = (o - mean) * lax.rsqrt(var + 1e-5) * g_ref[...] + b_ref[...]


def _tc_norm(a0, a1, na16, gamma, beta):
    return pl.pallas_call(
        _tc_norm_body,
        out_shape=jax.ShapeDtypeStruct((N, OUT), jnp.float32),
    )(a0, a1, na16, gamma, beta)


def kernel(node_attr, edge_index, edge_attr, edge_sh, W1, b1, W2, b2, gamma, beta):
    na16 = jnp.pad(node_attr, ((0, 0), (0, 2 * IN - IN)))
    src = edge_index[0]
    dst = edge_index[1]
    x = _sc_gather(na16, dst)
    rows = _tc_main(x, edge_sh, edge_attr, W1, b1.reshape(1, H), W2,
                    b2.reshape(1, K * OUT))
    acc = _sc_scatter(rows, src, jnp.zeros((N, WID), jnp.float32))
    return _tc_norm(acc[0], acc[1], na16, gamma.reshape(1, OUT),
                    beta.reshape(1, OUT))


# speedup vs baseline: 2.8778x; 1.1908x over previous
import functools

import numpy as np
import jax
import jax.numpy as jnp
from jax import lax
from jax.experimental import pallas as pl
from jax.experimental.pallas import tpu as pltpu
from jax.experimental.pallas import tpu_sc as plsc

N = 10000
E = 160000
IN = 8
SH = 4
OUT = 16
F = 64
H = 64
K = IN * SH
WID = 2 * OUT
SCALE = 1.0 / float(np.sqrt(IN * SH))

NC, NS = 2, 16
NW = NC * NS
EPW = E // NW
NPT = N // NS
CH = 1000

BE = 640

_c = np.arange(K * OUT)
_RX = ((_c // (SH * OUT))[None, :] == np.arange(2 * IN)[:, None]).astype(np.float32)
_RS = (((_c // OUT) % SH)[None, :] == np.arange(SH)[:, None]).astype(np.float32)


def _sc_gather(table, idx):
    mesh = plsc.VectorSubcoreMesh(core_axis_name="c", subcore_axis_name="s")

    @functools.partial(
        pl.kernel, mesh=mesh,
        out_type=jax.ShapeDtypeStruct((E, 2 * IN), jnp.float32),
        compiler_params=pltpu.CompilerParams(use_tc_tiling_on_sc=False),
        scratch_types=[
            pltpu.VMEM((EPW,), jnp.int32),
            pltpu.VMEM((EPW, 2 * IN), jnp.float32),
            pltpu.SemaphoreType.DMA,
        ],
    )
    def gk(table_hbm, idx_hbm, out_hbm, idx_v, rows_v, sem):
        wid = lax.axis_index("s") * NC + lax.axis_index("c")
        base = wid * EPW
        pltpu.sync_copy(idx_hbm.at[pl.ds(base, EPW)], idx_v)
        pltpu.async_copy(table_hbm.at[idx_v], rows_v, sem).wait()
        pltpu.sync_copy(rows_v, out_hbm.at[pl.ds(base, EPW)])

    return gk(table, idx)


def _sc_scatter(rows, src, zeros):
    mesh = plsc.VectorSubcoreMesh(core_axis_name="c", subcore_axis_name="s")

    @functools.partial(
        pl.kernel, mesh=mesh,
        out_type=jax.ShapeDtypeStruct((NC, N, WID), jnp.float32),
        compiler_params=pltpu.CompilerParams(use_tc_tiling_on_sc=False),
        scratch_types=[
            pltpu.VMEM((CH,), jnp.int32),
            pltpu.VMEM((CH, WID), jnp.float32),
            pltpu.VMEM_SHARED((N, WID), jnp.float32),
            pltpu.SemaphoreType.DMA,
        ],
    )
    def sk(rows_hbm, src_hbm, zeros_hbm, out_hbm, idx_v, rows_v, acc_sh, sem):
        c = lax.axis_index("c")
        s = lax.axis_index("s")
        pltpu.sync_copy(zeros_hbm.at[pl.ds(s * NPT, NPT)],
                        acc_sh.at[pl.ds(s * NPT, NPT)])
        plsc.subcore_barrier()
        base = (s * NC + c) * EPW

        def body(i, carry):
            off = base + i * CH
            pltpu.sync_copy(src_hbm.at[pl.ds(off, CH)], idx_v)
            pltpu.sync_copy(rows_hbm.at[pl.ds(off, CH)], rows_v)
            pltpu.sync_copy(rows_v, acc_sh.at[idx_v], add=True)
            return carry

        lax.fori_loop(0, EPW // CH, body, 0)
        plsc.subcore_barrier()
        pltpu.sync_copy(acc_sh.at[pl.ds(s * NPT, NPT)],
                        out_hbm.at[c, pl.ds(s * NPT, NPT)])

    return sk(rows, src, zeros)


def _dott(at, b):
    return lax.dot_general(at, b, dimension_numbers=(((0,), (0,)), ((), ())),
                           preferred_element_type=jnp.float32)


def _tc_main_body(x_ref, sht_ref, eat_ref, w1_ref, b1_ref, w2_ref, b2_ref,
                  rx_ref, rs_ref, out_ref):
    h = jnp.maximum(_dott(eat_ref[...], w1_ref[...]) + b1_ref[...], 0.0)
    w = jnp.dot(h, w2_ref[...], preferred_element_type=jnp.float32) + b2_ref[...]
    xb = jnp.dot(x_ref[...], rx_ref[...], preferred_element_type=jnp.float32)
    sb = _dott(sht_ref[...], rs_ref[...])
    p = xb * sb * w
    t = p[:, :256] + p[:, 256:]
    t = t[:, :128] + t[:, 128:]
    t = t[:, :64] + t[:, 64:]
    t = t[:, :32] + t[:, 32:]
    tp = (t[:, :16] + t[:, 16:]) * SCALE
    out_ref[...] = jnp.concatenate(
        [tp, jnp.ones((tp.shape[0], OUT), jnp.float32)], axis=1)


def _tc_main(x, sht, eat, w1, b1, w2, b2):
    nb = E // BE
    fixed = lambda i: (0, 0)
    return pl.pallas_call(
        _tc_main_body,
        grid=(nb,),
        in_specs=[
            pl.BlockSpec((BE, 2 * IN), lambda i: (i, 0)),
            pl.BlockSpec((SH, BE), lambda i: (0, i)),
            pl.BlockSpec((F, BE), lambda i: (0, i)),
            pl.BlockSpec((F, H), fixed),
            pl.BlockSpec((1, H), fixed),
            pl.BlockSpec((H, K * OUT), fixed),
            pl.BlockSpec((1, K * OUT), fixed),
            pl.BlockSpec((2 * IN, K * OUT), fixed),
            pl.BlockSpec((SH, K * OUT), fixed),
        ],
        out_specs=pl.BlockSpec((BE, WID), lambda i: (i, 0)),
        out_shape=jax.ShapeDtypeStruct((E, WID), jnp.float32),
    )(x, sht, eat, w1, b1, w2, b2, jnp.asarray(_RX), jnp.asarray(_RS))


def _tc_norm_body(a0_ref, a1_ref, na_ref, g_ref, b_ref, out_ref):
    a = a0_ref[...] + a1_ref[...]
    sums = a[:, :OUT]
    cnt = a[:, OUT:OUT + 1]
    o = sums / jnp.maximum(cnt, 1.0) + na_ref[...]
    mean = jnp.mean(o, axis=0, keepdims=True)
    var = jnp.mean((o - mean) ** 2, axis=0, keepdims=True)
    out_ref[...] = (o - mean) * lax.rsqrt(var + 1e-5) * g_ref[...] + b_ref[...]


def _tc_norm(a0, a1, na16, gamma, beta):
    return pl.pallas_call(
        _tc_norm_body,
        out_shape=jax.ShapeDtypeStruct((N, OUT), jnp.float32),
    )(a0, a1, na16, gamma, beta)


def kernel(node_attr, edge_index, edge_attr, edge_sh, W1, b1, W2, b2, gamma, beta):
    na16 = jnp.pad(node_attr, ((0, 0), (0, 2 * IN - IN)))
    src = edge_index[0]
    dst = edge_index[1]
    x = _sc_gather(na16, dst)
    rows = _tc_main(x, edge_sh.T, edge_attr.T, W1, b1.reshape(1, H), W2,
                    b2.reshape(1, K * OUT))
    acc = _sc_scatter(rows, src, jnp.zeros((N, WID), jnp.float32))
    return _tc_norm(acc[0], acc[1], na16, gamma.reshape(1, OUT),
                    beta.reshape(1, OUT))
